# baseline (speedup 1.0000x reference)
import functools

import jax
import jax.numpy as jnp
from jax import lax
from jax.experimental import pallas as pl
from jax.experimental.pallas import tpu as pltpu
from jax.experimental.pallas import tpu_sc as plsc

_K = 25
_MIN_CONF = 0.25
_BQ = 512
_CK = 128


def _prep_body(e_ref, c_ref, en_ref, eh_ref, el_ref, h_ref, mxhi_ref,
               mxlo_ref, np_ref):
    E = e_ref[...]
    en_ref[...] = jnp.sum(E * E, axis=1)[None, :]
    eh = E.astype(jnp.bfloat16)
    eh_ref[...] = eh
    el_ref[...] = (E - eh.astype(jnp.float32)).astype(jnp.bfloat16)

    C = c_ref[...]
    mx = jnp.max(C, axis=1, keepdims=True)
    col = lax.broadcasted_iota(jnp.int32, C.shape, 1)
    am = jnp.min(jnp.where(C == mx, col, C.shape[1]), axis=1, keepdims=True)
    H = (col == am).astype(jnp.bfloat16)
    h_ref[...] = H
    mxhi = mx.astype(jnp.bfloat16)
    mxhi_ref[...] = mxhi
    mxlo_ref[...] = (mx - mxhi.astype(jnp.float32)).astype(jnp.bfloat16)

    sel = (mx >= _MIN_CONF).astype(jnp.bfloat16)
    pop = lax.dot_general(sel, H, (((0,), (0,)), ((), ())),
                          preferred_element_type=jnp.float32)
    np_ref[0, 0] = jnp.sum((pop > 0.0).astype(jnp.float32))


def _tc_prep(encodings, categorical):
    B, D = encodings.shape
    nclust = categorical.shape[1]
    return pl.pallas_call(
        _prep_body,
        out_specs=[
            pl.BlockSpec((1, B), lambda: (0, 0)),
            pl.BlockSpec((B, D), lambda: (0, 0)),
            pl.BlockSpec((B, D), lambda: (0, 0)),
            pl.BlockSpec((B, nclust), lambda: (0, 0)),
            pl.BlockSpec((B, 1), lambda: (0, 0)),
            pl.BlockSpec((B, 1), lambda: (0, 0)),
            pl.BlockSpec((1, 1), lambda: (0, 0), memory_space=pltpu.SMEM),
        ],
        out_shape=[
            jax.ShapeDtypeStruct((1, B), jnp.float32),
            jax.ShapeDtypeStruct((B, D), jnp.bfloat16),
            jax.ShapeDtypeStruct((B, D), jnp.bfloat16),
            jax.ShapeDtypeStruct((B, nclust), jnp.bfloat16),
            jax.ShapeDtypeStruct((B, 1), jnp.bfloat16),
            jax.ShapeDtypeStruct((B, 1), jnp.bfloat16),
            jax.ShapeDtypeStruct((1, 1), jnp.float32),
        ],
    )(encodings, categorical)


def _main_body(en_ref, eh_ref, el_ref, h_ref, mxhi_ref, mxlo_ref, q_ref,
               qn_ref, idx_ref, ne_ref, sbuf_ref, qnorm_ref):
    dims = (((1,), (1,)), ((), ()))
    nq = q_ref.shape[0]
    nkeys = eh_ref.shape[0]
    nchunk = nkeys // _CK

    @pl.when(pl.program_id(0) == 0)
    def _():
        Q0 = q_ref[...]
        q0h = Q0.astype(jnp.bfloat16)
        q0l = (Q0 - q0h.astype(jnp.float32)).astype(jnp.bfloat16)
        E_h = eh_ref[...]
        E_l = el_ref[...]
        sbuf_ref[...] = (
            lax.dot_general(q0h, E_h, dims, preferred_element_type=jnp.float32)
            + lax.dot_general(q0h, E_l, dims, preferred_element_type=jnp.float32)
            + lax.dot_general(q0l, E_h, dims, preferred_element_type=jnp.float32))
        qnorm_ref[...] = jnp.sum(Q0 * Q0, axis=1, keepdims=True)

    qn_cur = qnorm_ref[...]
    d2 = jnp.maximum(qn_cur + en_ref[...] - 2.0 * sbuf_ref[...], 0.0)

    Q1 = qn_ref[...]
    q1h = Q1.astype(jnp.bfloat16)
    q1l = (Q1 - q1h.astype(jnp.float32)).astype(jnp.bfloat16)
    qnorm_ref[...] = jnp.sum(Q1 * Q1, axis=1, keepdims=True)

    dbits = lax.bitcast_convert_type(d2, jnp.int32)
    ones = jnp.ones((nkeys, 1), dtype=jnp.bfloat16)

    def _schunk(c):
        ehc = eh_ref[pl.ds(c * _CK, _CK), :]
        elc = el_ref[pl.ds(c * _CK, _CK), :]
        sbuf_ref[:, pl.ds(c * _CK, _CK)] = (
            lax.dot_general(q1h, ehc, dims, preferred_element_type=jnp.float32)
            + lax.dot_general(q1h, elc, dims, preferred_element_type=jnp.float32)
            + lax.dot_general(q1l, ehc, dims, preferred_element_type=jnp.float32))

    def bstep(i, p):
        _schunk(i)
        cand = p + (jnp.int32(1) << (30 - i))
        cmpb = (dbits < cand).astype(jnp.bfloat16)
        c = lax.dot_general(cmpb, ones, (((1,), (0,)), ((), ())),
                            preferred_element_type=jnp.float32)
        return jnp.where(c >= float(_K + 1), p, cand)

    p = lax.fori_loop(0, 31, bstep, jnp.zeros((nq, 1), jnp.int32))
    for c in range(31, nchunk):
        _schunk(c)
    mask = (dbits < p).astype(jnp.bfloat16)

    counts = lax.dot_general(mask, h_ref[...], (((1,), (0,)), ((), ())),
                             preferred_element_type=jnp.float32)
    total = jnp.sum(counts, axis=1, keepdims=True)
    bins = counts / total
    ent = -jnp.sum(bins * jnp.log(bins + 1e-5), axis=1)

    idx = idx_ref[...]
    rowid = lax.broadcasted_iota(jnp.int32, (idx.shape[0], nkeys), 1)
    P = (idx[:, None] == rowid).astype(jnp.bfloat16)
    mg = (lax.dot_general(P, mxhi_ref[...], (((1,), (0,)), ((), ())),
                          preferred_element_type=jnp.float32)
          + lax.dot_general(P, mxlo_ref[...], (((1,), (0,)), ((), ())),
                            preferred_element_type=jnp.float32))
    ne_ref[...] = ent * mg[:, 0]


def _tc_main(en, eh, el, h, mxhi, mxlo, q, idx):
    B, D = eh.shape
    nclust = h.shape[1]
    nsamp = q.shape[0]
    grid = nsamp // _BQ
    return pl.pallas_call(
        _main_body,
        grid=(grid,),
        in_specs=[
            pl.BlockSpec((1, B), lambda i: (0, 0)),
            pl.BlockSpec((B, D), lambda i: (0, 0)),
            pl.BlockSpec((B, D), lambda i: (0, 0)),
            pl.BlockSpec((B, nclust), lambda i: (0, 0)),
            pl.BlockSpec((B, 1), lambda i: (0, 0)),
            pl.BlockSpec((B, 1), lambda i: (0, 0)),
            pl.BlockSpec((_BQ, D), lambda i: (i, 0)),
            pl.BlockSpec((_BQ, D), lambda i: ((i + 1) % grid, 0)),
            pl.BlockSpec((_BQ,), lambda i: (i,)),
        ],
        out_specs=pl.BlockSpec((_BQ,), lambda i: (i,)),
        out_shape=jax.ShapeDtypeStruct((nsamp,), jnp.float32),
        scratch_shapes=[
            pltpu.VMEM((_BQ, B), jnp.float32),
            pltpu.VMEM((_BQ, 1), jnp.float32),
        ],
    )(en, eh, el, h, mxhi, mxlo, q, q, idx)


def _sc_gather(encodings, idx):
    B, D = encodings.shape
    nsamp = idx.shape[0]
    info = plsc.get_sparse_core_info()
    ncores = info.num_cores
    nw = ncores * info.num_subcores
    bpw = nsamp // nw
    mesh = plsc.VectorSubcoreMesh(core_axis_name="c", subcore_axis_name="s")

    @functools.partial(
        pl.kernel, mesh=mesh,
        out_type=jax.ShapeDtypeStruct((nsamp, D), jnp.float32),
        scratch_types=[
            pltpu.VMEM((bpw,), jnp.int32),
            pltpu.VMEM((bpw, D), jnp.float32),
            pltpu.SemaphoreType.DMA,
        ],
    )
    def k(enc, ih, qo, idx_v, rows_v, s1):
        wid = lax.axis_index("s") * ncores + lax.axis_index("c")
        base = wid * bpw
        pltpu.sync_copy(ih.at[pl.ds(base, bpw)], idx_v)
        pltpu.async_copy(enc.at[idx_v], rows_v, s1).wait()
        pltpu.sync_copy(rows_v, qo.at[pl.ds(base, bpw)])

    return k(encodings, idx)


def kernel(encodings, categorical, random_idxs):
    q = _sc_gather(encodings, random_idxs)
    en, eh, el, h, mxhi, mxlo, npop = _tc_prep(encodings, categorical)
    ne = _tc_main(en, eh, el, h, mxhi, mxlo, q, random_idxs)
    return encodings, ne, npop[0, 0]


# speedup vs baseline: 1.2343x; 1.2343x over previous
import functools

import jax
import jax.numpy as jnp
from jax import lax
from jax.experimental import pallas as pl
from jax.experimental.pallas import tpu as pltpu
from jax.experimental.pallas import tpu_sc as plsc

_K = 25
_MIN_CONF = 0.25
_BQ = 1024


def _prep_body(e_ref, c_ref, en_ref, eh_ref, el_ref, h_ref, mxhi_ref,
               mxlo_ref, np_ref):
    E = e_ref[...]
    en_ref[...] = jnp.sum(E * E, axis=1)[None, :]
    eh = E.astype(jnp.bfloat16)
    eh_ref[...] = eh
    el_ref[...] = (E - eh.astype(jnp.float32)).astype(jnp.bfloat16)

    C = c_ref[...]
    mx = jnp.max(C, axis=1, keepdims=True)
    col = lax.broadcasted_iota(jnp.int32, C.shape, 1)
    am = jnp.min(jnp.where(C == mx, col, C.shape[1]), axis=1, keepdims=True)
    H = (col == am).astype(jnp.bfloat16)
    h_ref[...] = H
    mxhi = mx.astype(jnp.bfloat16)
    mxhi_ref[...] = mxhi
    mxlo_ref[...] = (mx - mxhi.astype(jnp.float32)).astype(jnp.bfloat16)

    sel = (mx >= _MIN_CONF).astype(jnp.bfloat16)
    pop = lax.dot_general(sel, H, (((0,), (0,)), ((), ())),
                          preferred_element_type=jnp.float32)
    np_ref[0, 0] = jnp.sum((pop > 0.0).astype(jnp.float32))


def _tc_prep(encodings, categorical):
    B, D = encodings.shape
    nclust = categorical.shape[1]
    return pl.pallas_call(
        _prep_body,
        out_specs=[
            pl.BlockSpec((1, B), lambda: (0, 0)),
            pl.BlockSpec((B, D), lambda: (0, 0)),
            pl.BlockSpec((B, D), lambda: (0, 0)),
            pl.BlockSpec((B, nclust), lambda: (0, 0)),
            pl.BlockSpec((B, 1), lambda: (0, 0)),
            pl.BlockSpec((B, 1), lambda: (0, 0)),
            pl.BlockSpec((1, 1), lambda: (0, 0), memory_space=pltpu.SMEM),
        ],
        out_shape=[
            jax.ShapeDtypeStruct((1, B), jnp.float32),
            jax.ShapeDtypeStruct((B, D), jnp.bfloat16),
            jax.ShapeDtypeStruct((B, D), jnp.bfloat16),
            jax.ShapeDtypeStruct((B, nclust), jnp.bfloat16),
            jax.ShapeDtypeStruct((B, 1), jnp.bfloat16),
            jax.ShapeDtypeStruct((B, 1), jnp.bfloat16),
            jax.ShapeDtypeStruct((1, 1), jnp.float32),
        ],
    )(encodings, categorical)


def _main_body(en_ref, eh_ref, el_ref, h_ref, mxhi_ref, mxlo_ref, q_ref,
               idx_ref, ne_ref):
    Q = q_ref[...]
    qh = Q.astype(jnp.bfloat16)
    ql = (Q - qh.astype(jnp.float32)).astype(jnp.bfloat16)
    dims = (((1,), (1,)), ((), ()))
    S = (lax.dot_general(qh, eh_ref[...], dims, preferred_element_type=jnp.float32)
         + lax.dot_general(qh, el_ref[...], dims, preferred_element_type=jnp.float32)
         + lax.dot_general(ql, eh_ref[...], dims, preferred_element_type=jnp.float32))
    qn = jnp.sum(Q * Q, axis=1)
    d2 = jnp.maximum(qn[:, None] + en_ref[...] - 2.0 * S, 0.0)

    dbits = lax.bitcast_convert_type(d2, jnp.int32)
    nkeys = eh_ref.shape[0]
    ones = jnp.ones((nkeys, 1), dtype=jnp.bfloat16)

    def bstep(i, p):
        cand = p + (jnp.int32(1) << (30 - i))
        cmpb = (dbits < cand).astype(jnp.bfloat16)
        c = lax.dot_general(cmpb, ones, (((1,), (0,)), ((), ())),
                            preferred_element_type=jnp.float32)
        return jnp.where(c >= float(_K + 1), p, cand)

    p = lax.fori_loop(0, 31, bstep, jnp.zeros((Q.shape[0], 1), jnp.int32))
    mask = (dbits < p).astype(jnp.bfloat16)

    counts = lax.dot_general(mask, h_ref[...], (((1,), (0,)), ((), ())),
                             preferred_element_type=jnp.float32)
    total = jnp.sum(counts, axis=1, keepdims=True)
    bins = counts / total
    ent = -jnp.sum(bins * jnp.log(bins + 1e-5), axis=1)

    idx = idx_ref[...]
    rowid = lax.broadcasted_iota(jnp.int32, (idx.shape[0], nkeys), 1)
    P = (idx[:, None] == rowid).astype(jnp.bfloat16)
    mg = (lax.dot_general(P, mxhi_ref[...], (((1,), (0,)), ((), ())),
                          preferred_element_type=jnp.float32)
          + lax.dot_general(P, mxlo_ref[...], (((1,), (0,)), ((), ())),
                            preferred_element_type=jnp.float32))
    ne_ref[...] = ent * mg[:, 0]


def _tc_main(en, eh, el, h, mxhi, mxlo, q, idx):
    B, D = eh.shape
    nclust = h.shape[1]
    nsamp = q.shape[0]
    grid = nsamp // _BQ
    return pl.pallas_call(
        _main_body,
        grid=(grid,),
        in_specs=[
            pl.BlockSpec((1, B), lambda i: (0, 0)),
            pl.BlockSpec((B, D), lambda i: (0, 0)),
            pl.BlockSpec((B, D), lambda i: (0, 0)),
            pl.BlockSpec((B, nclust), lambda i: (0, 0)),
            pl.BlockSpec((B, 1), lambda i: (0, 0)),
            pl.BlockSpec((B, 1), lambda i: (0, 0)),
            pl.BlockSpec((_BQ, D), lambda i: (i, 0)),
            pl.BlockSpec((_BQ,), lambda i: (i,)),
        ],
        out_specs=pl.BlockSpec((_BQ,), lambda i: (i,)),
        out_shape=jax.ShapeDtypeStruct((nsamp,), jnp.float32),
    )(en, eh, el, h, mxhi, mxlo, q, idx)


def _sc_gather(encodings, idx):
    B, D = encodings.shape
    nsamp = idx.shape[0]
    info = plsc.get_sparse_core_info()
    ncores = info.num_cores
    nw = ncores * info.num_subcores
    bpw = nsamp // nw
    mesh = plsc.VectorSubcoreMesh(core_axis_name="c", subcore_axis_name="s")

    @functools.partial(
        pl.kernel, mesh=mesh,
        out_type=jax.ShapeDtypeStruct((nsamp, D), jnp.float32),
        scratch_types=[
            pltpu.VMEM((bpw,), jnp.int32),
            pltpu.VMEM((bpw, D), jnp.float32),
            pltpu.SemaphoreType.DMA,
        ],
    )
    def k(enc, ih, qo, idx_v, rows_v, s1):
        wid = lax.axis_index("s") * ncores + lax.axis_index("c")
        base = wid * bpw
        pltpu.sync_copy(ih.at[pl.ds(base, bpw)], idx_v)
        pltpu.async_copy(enc.at[idx_v], rows_v, s1).wait()
        pltpu.sync_copy(rows_v, qo.at[pl.ds(base, bpw)])

    return k(encodings, idx)


def kernel(encodings, categorical, random_idxs):
    q = _sc_gather(encodings, random_idxs)
    en, eh, el, h, mxhi, mxlo, npop = _tc_prep(encodings, categorical)
    ne = _tc_main(en, eh, el, h, mxhi, mxlo, q, random_idxs)
    return encodings, ne, npop[0, 0]
